# baseline (speedup 1.0000x reference)
import functools

import jax
import jax.numpy as jnp
from jax import lax
from jax.experimental import pallas as pl
from jax.experimental.pallas import tpu as pltpu
from jax.experimental.pallas import tpu_sc as plsc

_B, _S, _D = 1024, 200, 64
_NT = 20
_GOFF = 16
_GLEN = _S - _GOFF
_TAIL = _S - _NT
_C0, _C1 = 96, _GLEN - 96


@functools.cache
def _build(nc: int, ns: int):
    nw = nc * ns
    bpw = _B // nw
    mesh = plsc.VectorSubcoreMesh(
        core_axis_name="c", subcore_axis_name="s",
        num_cores=nc, num_subcores=ns)

    @functools.partial(
        pl.kernel,
        out_type=jax.ShapeDtypeStruct((_B, _S, _D), jnp.float32),
        mesh=mesh,
        scratch_types=[
            pltpu.VMEM((_GLEN,), jnp.int32),
            pltpu.VMEM((_GLEN, _D), jnp.float32),
            pltpu.VMEM((_NT, _D), jnp.float32),
            pltpu.SemaphoreType.DMA,
        ],
        compiler_params=pltpu.CompilerParams(use_tc_tiling_on_sc=False),
    )
    def soft_embed(tokens_hbm, wte_hbm, learned_hbm, out_hbm,
                   tok_v, rows_v, learned_v, sem):
        wid = lax.axis_index("s") * nc + lax.axis_index("c")
        base = wid * bpw
        pltpu.sync_copy(learned_hbm, learned_v)

        def body(i, carry):
            b = base + i
            pltpu.sync_copy(tokens_hbm.at[b, pl.ds(_GOFF, _GLEN)], tok_v)
            cp0 = pltpu.async_copy(
                wte_hbm.at[tok_v.at[pl.ds(0, _C0)]],
                rows_v.at[pl.ds(0, _C0)], sem)
            cp1 = pltpu.async_copy(
                wte_hbm.at[tok_v.at[pl.ds(_C0, _C1)]],
                rows_v.at[pl.ds(_C0, _C1)], sem)
            pltpu.sync_copy(learned_v, out_hbm.at[b, pl.ds(0, _NT)])
            cp0.wait()
            cp1.wait()
            pltpu.sync_copy(rows_v.at[pl.ds(_NT - _GOFF, _TAIL)],
                            out_hbm.at[b, pl.ds(_NT, _TAIL)])
            return carry

        lax.fori_loop(0, bpw, body, 0)

    return soft_embed


def kernel(tokens, wte_weight, learned_embedding):
    info = plsc.get_sparse_core_info()
    k = _build(info.num_cores, info.num_subcores)
    return k(tokens.astype(jnp.int32), wte_weight, learned_embedding)
